# baseline (speedup 1.0000x reference)
import functools

import jax
import jax.numpy as jnp
from jax import lax
from jax.experimental import pallas as pl
from jax.experimental.pallas import tpu as pltpu
from jax.experimental.pallas import tpu_sc as plsc

CH = 128
BR = 512
N_TILES = 16
FH = 128


def _ceil_to(a, m):
    return (a + m - 1) // m * m


def _make_deg(npad, nch_tile):
    half = npad // 2
    rpt = npad // N_TILES
    hrpt = half // N_TILES
    mesh = plsc.VectorSubcoreMesh(core_axis_name="c", subcore_axis_name="s")

    @functools.partial(
        pl.kernel,
        out_type=jax.ShapeDtypeStruct((npad, 8), jnp.float32),
        mesh=mesh,
        scratch_types=[
            pltpu.VMEM((nch_tile, CH), jnp.int32),
            pltpu.VMEM((CH, 8), jnp.float32),
            pltpu.VMEM_SHARED((npad + 1, 8), jnp.float32),
        ],
    )
    def deg_kernel(dst2d, ones_hbm, deg_out, dst_v, ones_v, acc):
        c = lax.axis_index("c")
        s = lax.axis_index("s")
        pltpu.sync_copy(dst2d.at[pl.ds(s * nch_tile, nch_tile)], dst_v)
        pltpu.sync_copy(ones_hbm.at[pl.ds(0, CH)], ones_v)
        pltpu.sync_copy(ones_hbm.at[pl.ds(s * rpt, rpt)],
                        acc.at[pl.ds(s * rpt, rpt)])
        plsc.subcore_barrier()

        def body(j, carry):
            pltpu.sync_copy(ones_v, acc.at[dst_v.at[j]], add=True)
            return carry

        lax.fori_loop(0, nch_tile, body, 0)
        plsc.subcore_barrier()
        r0 = c * half + s * hrpt
        pltpu.sync_copy(acc.at[pl.ds(r0, hrpt)], deg_out.at[pl.ds(r0, hrpt)])

    return deg_kernel


def _make_prop(npad, nch_tile):
    rpt = npad // N_TILES
    mesh = plsc.VectorSubcoreMesh(core_axis_name="c", subcore_axis_name="s")

    nch_p = nch_tile // 2

    @functools.partial(
        pl.kernel,
        out_type=jax.ShapeDtypeStruct((2 * npad, FH), jnp.float32),
        mesh=mesh,
        scratch_types=[
            pltpu.VMEM((nch_p, CH), jnp.int32),
            pltpu.VMEM((nch_p, CH), jnp.int32),
            pltpu.VMEM((CH, FH), jnp.float32),
            pltpu.VMEM((CH, FH), jnp.float32),
            pltpu.VMEM_SHARED((npad + 1, FH), jnp.float32),
            pltpu.SemaphoreType.DMA,
            pltpu.SemaphoreType.DMA,
        ],
    )
    def prop_kernel(y_cat, src3, dst2d, out_cat,
                    src_v, dst_v, buf0, buf1, acc, sem0, sem1):
        c = lax.axis_index("c")
        s = lax.axis_index("s")
        r0 = s * rpt
        base = c * npad
        nch_rows = N_TILES * nch_tile
        npairs = nch_p // 2

        pltpu.sync_copy(y_cat.at[pl.ds(base + r0, rpt)], acc.at[pl.ds(r0, rpt)])
        plsc.subcore_barrier()

        for p in range(2):
            e0 = s * nch_tile + p * nch_p
            pltpu.sync_copy(src3.at[pl.ds(c * nch_rows + e0, nch_p)], src_v)
            pltpu.sync_copy(dst2d.at[pl.ds(e0, nch_p)], dst_v)
            def body(jj, carry):
                j0 = 2 * jj
                j1 = j0 + 1
                g0 = pltpu.async_copy(y_cat.at[src_v.at[j0]], buf0, sem0)
                g0.wait()
                g1 = pltpu.async_copy(y_cat.at[src_v.at[j1]], buf1, sem1)
                pltpu.sync_copy(buf0, acc.at[dst_v.at[j0]], add=True)
                g1.wait()
                pltpu.sync_copy(buf1, acc.at[dst_v.at[j1]], add=True)
                return carry

            lax.fori_loop(0, npairs, body, 0)

        plsc.subcore_barrier()
        pltpu.sync_copy(acc.at[pl.ds(r0, rpt)],
                        out_cat.at[pl.ds(base + r0, rpt)])

    return prop_kernel


def _make_scale(npad):
    nb = npad // BR

    def body(deg_ref, x_ref, o_ref):
        dinv = lax.rsqrt(deg_ref[:, 0:1])
        o_ref[...] = x_ref[...] * dinv

    return pl.pallas_call(
        body,
        grid=(2, nb),
        in_specs=[pl.BlockSpec((BR, 8), lambda j, i: (i, 0)),
                  pl.BlockSpec((BR, FH), lambda j, i: (i, j))],
        out_specs=pl.BlockSpec((BR, FH), lambda j, i: (j * nb + i, 0)),
        out_shape=jax.ShapeDtypeStruct((2 * npad, FH), jnp.float32),
    )


def _make_layer1(npad, fout):
    nb = npad // BR

    def body(deg_ref, lo_ref, hi_ref, w_ref, b_ref, o_ref):
        j = pl.program_id(0)
        dinv = lax.rsqrt(deg_ref[:, 0:1])
        sfull = jnp.concatenate([lo_ref[...], hi_ref[...]], axis=1) * dinv
        h = jnp.dot(sfull, w_ref[...], preferred_element_type=jnp.float32)
        h = jnp.maximum(h + b_ref[...], 0.0) * dinv
        o_ref[...] = jnp.where(j == 0, h[:, :FH], h[:, FH:])

    return pl.pallas_call(
        body,
        grid=(2, nb),
        in_specs=[pl.BlockSpec((BR, 8), lambda j, i: (i, 0)),
                  pl.BlockSpec((BR, FH), lambda j, i: (i, 0)),
                  pl.BlockSpec((BR, FH), lambda j, i: (nb + i, 0)),
                  pl.BlockSpec((2 * FH, fout), lambda j, i: (0, 0)),
                  pl.BlockSpec((1, fout), lambda j, i: (0, 0))],
        out_specs=pl.BlockSpec((BR, FH), lambda j, i: (j * nb + i, 0)),
        out_shape=jax.ShapeDtypeStruct((2 * npad, FH), jnp.float32),
    )


def _make_layer2(npad, fout):
    nb = npad // BR

    def body(deg_ref, lo_ref, hi_ref, w_ref, b_ref, o_ref):
        dinv = lax.rsqrt(deg_ref[:, 0:1])
        sfull = jnp.concatenate([lo_ref[...], hi_ref[...]], axis=1) * dinv
        h = jnp.dot(sfull, w_ref[...], preferred_element_type=jnp.float32)
        o_ref[...] = jnp.maximum(h + b_ref[...], 0.0)

    return pl.pallas_call(
        body,
        grid=(nb,),
        in_specs=[pl.BlockSpec((BR, 8), lambda i: (i, 0)),
                  pl.BlockSpec((BR, FH), lambda i: (i, 0)),
                  pl.BlockSpec((BR, FH), lambda i: (nb + i, 0)),
                  pl.BlockSpec((2 * FH, fout), lambda i: (0, 0)),
                  pl.BlockSpec((1, fout), lambda i: (0, 0))],
        out_specs=pl.BlockSpec((BR, fout), lambda i: (i, 0)),
        out_shape=jax.ShapeDtypeStruct((npad, fout), jnp.float32),
    )


def kernel(x, edge_index, W1, b1, W2, b2):
    n, f = x.shape
    e = edge_index.shape[1]
    npad = _ceil_to(n, 1024)
    epad = _ceil_to(e, CH * N_TILES * 8)
    nch_tile = epad // (CH * N_TILES)

    ei = edge_index.astype(jnp.int32)
    src = jnp.pad(ei[0], (0, epad - e))
    dst = jnp.pad(ei[1], (0, epad - e), constant_values=npad)
    src2d = src.reshape(-1, CH)
    src3 = jnp.concatenate([src2d, src2d + npad], axis=0)
    dst2d = dst.reshape(-1, CH)
    x_p = jnp.pad(x, ((0, npad - n), (0, 0)))
    ones = jnp.ones((npad, 8), jnp.float32)

    deg_k = _make_deg(npad, nch_tile)
    prop_k = _make_prop(npad, nch_tile)
    scale_k = _make_scale(npad)
    layer1_k = _make_layer1(npad, f)
    layer2_k = _make_layer2(npad, W2.shape[1])

    deg = deg_k(dst2d, ones)
    y1 = scale_k(deg, x_p)
    s1 = prop_k(y1, src3, dst2d)
    y2 = layer1_k(deg, s1, s1, W1, b1.reshape(1, -1))
    s2 = prop_k(y2, src3, dst2d)
    out = layer2_k(deg, s2, s2, W2, b2.reshape(1, -1))
    return out[:n]


# speedup vs baseline: 1.0566x; 1.0566x over previous
import functools

import jax
import jax.numpy as jnp
from jax import lax
from jax.experimental import pallas as pl
from jax.experimental.pallas import tpu as pltpu
from jax.experimental.pallas import tpu_sc as plsc

CH = 128
BR = 512
N_TILES = 16
FH = 128


def _ceil_to(a, m):
    return (a + m - 1) // m * m


def _make_deg(npad, nch_tile):
    half = npad // 2
    rpt = npad // N_TILES
    hrpt = half // N_TILES
    mesh = plsc.VectorSubcoreMesh(core_axis_name="c", subcore_axis_name="s")

    @functools.partial(
        pl.kernel,
        out_type=jax.ShapeDtypeStruct((npad, 8), jnp.float32),
        mesh=mesh,
        scratch_types=[
            pltpu.VMEM((nch_tile, CH), jnp.int32),
            pltpu.VMEM((CH, 8), jnp.float32),
            pltpu.VMEM_SHARED((npad + 1, 8), jnp.float32),
        ],
    )
    def deg_kernel(dst2d, ones_hbm, deg_out, dst_v, ones_v, acc):
        c = lax.axis_index("c")
        s = lax.axis_index("s")
        pltpu.sync_copy(dst2d.at[pl.ds(s * nch_tile, nch_tile)], dst_v)
        pltpu.sync_copy(ones_hbm.at[pl.ds(0, CH)], ones_v)
        pltpu.sync_copy(ones_hbm.at[pl.ds(s * rpt, rpt)],
                        acc.at[pl.ds(s * rpt, rpt)])
        plsc.subcore_barrier()

        def body(j, carry):
            pltpu.sync_copy(ones_v, acc.at[dst_v.at[j]], add=True)
            return carry

        lax.fori_loop(0, nch_tile, body, 0)
        plsc.subcore_barrier()
        r0 = c * half + s * hrpt
        pltpu.sync_copy(acc.at[pl.ds(r0, hrpt)], deg_out.at[pl.ds(r0, hrpt)])

    return deg_kernel


def _make_prop(npad, nch_tile):
    rpt = npad // N_TILES
    mesh = plsc.VectorSubcoreMesh(core_axis_name="c", subcore_axis_name="s")

    @functools.partial(
        pl.kernel,
        out_type=jax.ShapeDtypeStruct((2 * npad, FH), jnp.float32),
        mesh=mesh,
        scratch_types=[
            pltpu.VMEM((nch_tile, CH), jnp.int32),
            pltpu.VMEM((nch_tile, CH), jnp.int32),
            pltpu.VMEM((CH, FH), jnp.float32),
            pltpu.VMEM_SHARED((npad + 1, FH), jnp.float32),
            pltpu.SemaphoreType.DMA,
        ],
    )
    def prop_kernel(y_cat, src3, dst2d, out_cat,
                    src_v, dst_v, buf0, acc, sem0):
        c = lax.axis_index("c")
        s = lax.axis_index("s")
        r0 = s * rpt
        base = c * npad
        nch_rows = N_TILES * nch_tile

        pltpu.sync_copy(y_cat.at[pl.ds(base + r0, rpt)], acc.at[pl.ds(r0, rpt)])
        e0 = s * nch_tile
        pltpu.sync_copy(src3.at[pl.ds(c * nch_rows + e0, nch_tile)], src_v)
        pltpu.sync_copy(dst2d.at[pl.ds(e0, nch_tile)], dst_v)
        plsc.subcore_barrier()

        def body(j, carry):
            pltpu.async_copy(y_cat.at[src_v.at[j]], buf0, sem0).wait()
            pltpu.sync_copy(buf0, acc.at[dst_v.at[j]], add=True)
            return carry

        lax.fori_loop(0, nch_tile, body, 0)

        plsc.subcore_barrier()
        pltpu.sync_copy(acc.at[pl.ds(r0, rpt)],
                        out_cat.at[pl.ds(base + r0, rpt)])

    return prop_kernel


def _make_scale(npad):
    nb = npad // BR

    def body(deg_ref, x_ref, o_ref):
        dinv = lax.rsqrt(deg_ref[:, 0:1])
        o_ref[...] = x_ref[...] * dinv

    return pl.pallas_call(
        body,
        grid=(2, nb),
        in_specs=[pl.BlockSpec((BR, 8), lambda j, i: (i, 0)),
                  pl.BlockSpec((BR, FH), lambda j, i: (i, j))],
        out_specs=pl.BlockSpec((BR, FH), lambda j, i: (j * nb + i, 0)),
        out_shape=jax.ShapeDtypeStruct((2 * npad, FH), jnp.float32),
    )


def _make_layer1(npad, fout):
    nb = npad // BR

    def body(deg_ref, lo_ref, hi_ref, w_ref, b_ref, o_ref):
        j = pl.program_id(0)
        dinv = lax.rsqrt(deg_ref[:, 0:1])
        sfull = jnp.concatenate([lo_ref[...], hi_ref[...]], axis=1) * dinv
        h = jnp.dot(sfull, w_ref[...], preferred_element_type=jnp.float32)
        h = jnp.maximum(h + b_ref[...], 0.0) * dinv
        o_ref[...] = jnp.where(j == 0, h[:, :FH], h[:, FH:])

    return pl.pallas_call(
        body,
        grid=(2, nb),
        in_specs=[pl.BlockSpec((BR, 8), lambda j, i: (i, 0)),
                  pl.BlockSpec((BR, FH), lambda j, i: (i, 0)),
                  pl.BlockSpec((BR, FH), lambda j, i: (nb + i, 0)),
                  pl.BlockSpec((2 * FH, fout), lambda j, i: (0, 0)),
                  pl.BlockSpec((1, fout), lambda j, i: (0, 0))],
        out_specs=pl.BlockSpec((BR, FH), lambda j, i: (j * nb + i, 0)),
        out_shape=jax.ShapeDtypeStruct((2 * npad, FH), jnp.float32),
    )


def _make_layer2(npad, fout, nrows_out):
    nb = npad // BR

    def body(deg_ref, lo_ref, hi_ref, w_ref, b_ref, o_ref):
        dinv = lax.rsqrt(deg_ref[:, 0:1])
        sfull = jnp.concatenate([lo_ref[...], hi_ref[...]], axis=1) * dinv
        h = jnp.dot(sfull, w_ref[...], preferred_element_type=jnp.float32)
        o_ref[...] = jnp.maximum(h + b_ref[...], 0.0)

    return pl.pallas_call(
        body,
        grid=(nb,),
        in_specs=[pl.BlockSpec((BR, 8), lambda i: (i, 0)),
                  pl.BlockSpec((BR, FH), lambda i: (i, 0)),
                  pl.BlockSpec((BR, FH), lambda i: (nb + i, 0)),
                  pl.BlockSpec((2 * FH, fout), lambda i: (0, 0)),
                  pl.BlockSpec((1, fout), lambda i: (0, 0))],
        out_specs=pl.BlockSpec((BR, fout), lambda i: (i, 0)),
        out_shape=jax.ShapeDtypeStruct((nrows_out, fout), jnp.float32),
    )


def kernel(x, edge_index, W1, b1, W2, b2):
    n, f = x.shape
    e = edge_index.shape[1]
    npad = _ceil_to(n, 1024)
    epad = _ceil_to(e, CH * N_TILES * 8)
    nch_tile = epad // (CH * N_TILES)

    ei = edge_index.astype(jnp.int32)
    src = jnp.pad(ei[0], (0, epad - e))
    dst = jnp.pad(ei[1], (0, epad - e), constant_values=npad)
    src2d = src.reshape(-1, CH)
    src3 = jnp.concatenate([src2d, src2d + npad], axis=0)
    dst2d = dst.reshape(-1, CH)
    x_p = jnp.pad(x, ((0, npad - n), (0, 0)))
    ones = jnp.ones((npad, 8), jnp.float32)

    deg_k = _make_deg(npad, nch_tile)
    prop_k = _make_prop(npad, nch_tile)
    scale_k = _make_scale(npad)
    layer1_k = _make_layer1(npad, f)
    layer2_k = _make_layer2(npad, W2.shape[1], n)

    deg = deg_k(dst2d, ones)
    y1 = scale_k(deg, x_p)
    s1 = prop_k(y1, src3, dst2d)
    y2 = layer1_k(deg, s1, s1, W1, b1.reshape(1, -1))
    s2 = prop_k(y2, src3, dst2d)
    return layer2_k(deg, s2, s2, W2, b2.reshape(1, -1))


# speedup vs baseline: 1.0662x; 1.0091x over previous
import functools

import jax
import jax.numpy as jnp
from jax import lax
from jax.experimental import pallas as pl
from jax.experimental.pallas import tpu as pltpu
from jax.experimental.pallas import tpu_sc as plsc

CH = 128
N_TILES = 16
FH = 128


def _ceil_to(a, m):
    return (a + m - 1) // m * m


def _make_deg(npad, nch_tile):
    half = npad // 2
    rpt = npad // N_TILES
    hrpt = half // N_TILES
    mesh = plsc.VectorSubcoreMesh(core_axis_name="c", subcore_axis_name="s")

    @functools.partial(
        pl.kernel,
        out_type=jax.ShapeDtypeStruct((npad, 8), jnp.float32),
        mesh=mesh,
        scratch_types=[
            pltpu.VMEM((nch_tile, CH), jnp.int32),
            pltpu.VMEM((CH, 8), jnp.float32),
            pltpu.VMEM_SHARED((npad + 1, 8), jnp.float32),
        ],
    )
    def deg_kernel(dst2d, ones_hbm, deg_out, dst_v, ones_v, acc):
        c = lax.axis_index("c")
        s = lax.axis_index("s")
        pltpu.sync_copy(dst2d.at[pl.ds(s * nch_tile, nch_tile)], dst_v)
        pltpu.sync_copy(ones_hbm.at[pl.ds(0, CH)], ones_v)
        pltpu.sync_copy(ones_hbm.at[pl.ds(s * rpt, rpt)],
                        acc.at[pl.ds(s * rpt, rpt)])
        plsc.subcore_barrier()

        def body(j, carry):
            pltpu.sync_copy(ones_v, acc.at[dst_v.at[j]], add=True)
            return carry

        lax.fori_loop(0, nch_tile, body, 0)
        plsc.subcore_barrier()
        r0 = c * half + s * hrpt
        pltpu.sync_copy(acc.at[pl.ds(r0, hrpt)], deg_out.at[pl.ds(r0, hrpt)])

    return deg_kernel


def _make_prop(npad, nch_tile):
    rpt = npad // N_TILES
    mesh = plsc.VectorSubcoreMesh(core_axis_name="c", subcore_axis_name="s")

    @functools.partial(
        pl.kernel,
        out_type=jax.ShapeDtypeStruct((2 * npad, FH), jnp.float32),
        mesh=mesh,
        scratch_types=[
            pltpu.VMEM((nch_tile, CH), jnp.int32),
            pltpu.VMEM((nch_tile, CH), jnp.int32),
            pltpu.VMEM((CH, FH), jnp.float32),
            pltpu.VMEM_SHARED((npad + 1, FH), jnp.float32),
            pltpu.SemaphoreType.DMA,
        ],
    )
    def prop_kernel(y_cat, src3, dst2d, out_cat,
                    src_v, dst_v, buf0, acc, sem0):
        c = lax.axis_index("c")
        s = lax.axis_index("s")
        r0 = s * rpt
        o0 = s * (2 * rpt) + c * rpt
        nch_rows = N_TILES * nch_tile

        pltpu.sync_copy(y_cat.at[pl.ds(o0, rpt)], acc.at[pl.ds(r0, rpt)])
        e0 = s * nch_tile
        pltpu.sync_copy(src3.at[pl.ds(c * nch_rows + e0, nch_tile)], src_v)
        pltpu.sync_copy(dst2d.at[pl.ds(e0, nch_tile)], dst_v)
        plsc.subcore_barrier()

        def body(j, carry):
            pltpu.async_copy(y_cat.at[src_v.at[j]], buf0, sem0).wait()
            pltpu.sync_copy(buf0, acc.at[dst_v.at[j]], add=True)
            return carry

        lax.fori_loop(0, nch_tile, body, 0)

        plsc.subcore_barrier()
        pltpu.sync_copy(acc.at[pl.ds(r0, rpt)],
                        out_cat.at[pl.ds(o0, rpt)])

    return prop_kernel


def _make_scale(npad):
    br = npad // N_TILES

    def body(deg_ref, x_ref, o_ref):
        dinv = lax.rsqrt(deg_ref[:, 0:1])
        t = x_ref[...] * dinv
        o_ref[...] = jnp.concatenate([t[:, :FH], t[:, FH:]], axis=0)

    return pl.pallas_call(
        body,
        grid=(N_TILES,),
        in_specs=[pl.BlockSpec((br, 8), lambda i: (i, 0)),
                  pl.BlockSpec((br, 2 * FH), lambda i: (i, 0))],
        out_specs=pl.BlockSpec((2 * br, FH), lambda i: (i, 0)),
        out_shape=jax.ShapeDtypeStruct((2 * npad, FH), jnp.float32),
    )


def _make_layer1(npad, fout):
    br = npad // N_TILES

    def body(deg_ref, lo_ref, hi_ref, w_ref, b_ref, o_ref):
        dinv = lax.rsqrt(deg_ref[:, 0:1])
        sfull = jnp.concatenate([lo_ref[...], hi_ref[...]], axis=1) * dinv
        h = jnp.dot(sfull, w_ref[...], preferred_element_type=jnp.float32)
        h = jnp.maximum(h + b_ref[...], 0.0) * dinv
        o_ref[...] = jnp.concatenate([h[:, :FH], h[:, FH:]], axis=0)

    return pl.pallas_call(
        body,
        grid=(N_TILES,),
        in_specs=[pl.BlockSpec((br, 8), lambda i: (i, 0)),
                  pl.BlockSpec((br, FH), lambda i: (2 * i, 0)),
                  pl.BlockSpec((br, FH), lambda i: (2 * i + 1, 0)),
                  pl.BlockSpec((2 * FH, fout), lambda i: (0, 0)),
                  pl.BlockSpec((1, fout), lambda i: (0, 0))],
        out_specs=pl.BlockSpec((2 * br, FH), lambda i: (i, 0)),
        out_shape=jax.ShapeDtypeStruct((2 * npad, FH), jnp.float32),
    )


def _make_layer2(npad, fout, nrows_out):
    br = npad // N_TILES

    def body(deg_ref, lo_ref, hi_ref, w_ref, b_ref, o_ref):
        dinv = lax.rsqrt(deg_ref[:, 0:1])
        sfull = jnp.concatenate([lo_ref[...], hi_ref[...]], axis=1) * dinv
        h = jnp.dot(sfull, w_ref[...], preferred_element_type=jnp.float32)
        o_ref[...] = jnp.maximum(h + b_ref[...], 0.0)

    return pl.pallas_call(
        body,
        grid=(N_TILES,),
        in_specs=[pl.BlockSpec((br, 8), lambda i: (i, 0)),
                  pl.BlockSpec((br, FH), lambda i: (2 * i, 0)),
                  pl.BlockSpec((br, FH), lambda i: (2 * i + 1, 0)),
                  pl.BlockSpec((2 * FH, fout), lambda i: (0, 0)),
                  pl.BlockSpec((1, fout), lambda i: (0, 0))],
        out_specs=pl.BlockSpec((br, fout), lambda i: (i, 0)),
        out_shape=jax.ShapeDtypeStruct((nrows_out, fout), jnp.float32),
    )


def kernel(x, edge_index, W1, b1, W2, b2):
    n, f = x.shape
    e = edge_index.shape[1]
    npad = _ceil_to(n, 1024)
    epad = _ceil_to(e, CH * N_TILES * 8)
    nch_tile = epad // (CH * N_TILES)

    ei = edge_index.astype(jnp.int32)
    src = jnp.pad(ei[0], (0, epad - e))
    dst = jnp.pad(ei[1], (0, epad - e), constant_values=npad)
    src2d = src.reshape(-1, CH)
    br = npad // N_TILES
    lo = (src2d // br) * (2 * br) + src2d % br
    src3 = jnp.concatenate([lo, lo + br], axis=0)
    dst2d = dst.reshape(-1, CH)
    x_p = jnp.pad(x, ((0, npad - n), (0, 0)))
    ones = jnp.ones((npad, 8), jnp.float32)

    deg_k = _make_deg(npad, nch_tile)
    prop_k = _make_prop(npad, nch_tile)
    scale_k = _make_scale(npad)
    layer1_k = _make_layer1(npad, f)
    layer2_k = _make_layer2(npad, W2.shape[1], n)

    deg = deg_k(dst2d, ones)
    y1 = scale_k(deg, x_p)
    s1 = prop_k(y1, src3, dst2d)
    y2 = layer1_k(deg, s1, s1, W1, b1.reshape(1, -1))
    s2 = prop_k(y2, src3, dst2d)
    return layer2_k(deg, s2, s2, W2, b2.reshape(1, -1))
